# Initial kernel scaffold; baseline (speedup 1.0000x reference)
import jax
import jax.numpy as jnp
from jax.experimental import pallas as pl


def kernel(user_emb, item_emb, ui_edge_index, ui_edge_weight, ln_weight, ln_bias):
    raise NotImplementedError("write your pallas kernel here")


# speedup vs baseline: 3.6342x; 3.6342x over previous
import functools

import jax
import jax.numpy as jnp
from jax import lax
from jax.experimental import pallas as pl
from jax.experimental.pallas import tpu as pltpu
from jax.experimental.pallas import tpu_sc as plsc

N_USERS = 10000
N_ITEMS = 10000
D = 128
E = 320000
N_LAYERS = 2
LANES = 16
NTILES = 16
CHUNK = 128
CPT = 160
SUP = 16
N_SUP = CPT // SUP
E_PAD = NTILES * CPT * CHUNK
ROW_BLK = 80
N_BLKS = N_USERS // ROW_BLK
BLKS_PER_TILE = (N_BLKS + NTILES - 1) // NTILES

_mesh = plsc.VectorSubcoreMesh(core_axis_name="c", subcore_axis_name="s")

_full = jax.ShapeDtypeStruct((N_USERS, D), jnp.float32)


@functools.partial(
    pl.kernel,
    mesh=_mesh,
    out_type=[_full, _full],
    scratch_types=[
        pltpu.VMEM((SUP, CHUNK), jnp.int32),
        pltpu.VMEM((SUP, CHUNK), jnp.int32),
        pltpu.VMEM((SUP, CHUNK), jnp.float32),
        pltpu.VMEM((CHUNK, D), jnp.float32),
        pltpu.VMEM_SHARED((N_USERS, D), jnp.float32),
        pltpu.SemaphoreType.DMA,
    ],
)
def _sc_layer(u_in, i_in, rows2d, cols2d, w2d,
              ua_out, ia_out,
              gixs, sixs, wvs, gbuf, acc, gsem):
    cid = lax.axis_index("c")
    sid = lax.axis_index("s")
    tile_chunk0 = pl.multiple_of(sid * CPT, CPT)

    def for_owned_blocks(fn):
        def t_body(t, _):
            blk = sid + t * NTILES

            @pl.when(blk < N_BLKS)
            def _():
                fn(pl.multiple_of(blk * ROW_BLK, ROW_BLK))
            return 0

        lax.fori_loop(0, BLKS_PER_TILE, t_body, 0)

    def edge_pass(base_tbl, src_tbl, out_tbl, gsrc2d, gdst2d):
        for_owned_blocks(
            lambda b: pltpu.sync_copy(base_tbl.at[pl.ds(b, ROW_BLK)],
                                      acc.at[pl.ds(b, ROW_BLK)]))
        plsc.subcore_barrier()

        def sup_body(s, _):
            sbase = pl.multiple_of(tile_chunk0 + s * SUP, SUP)
            pltpu.sync_copy(gsrc2d.at[pl.ds(sbase, SUP)], gixs)
            pltpu.sync_copy(gdst2d.at[pl.ds(sbase, SUP)], sixs)
            pltpu.sync_copy(w2d.at[pl.ds(sbase, SUP)], wvs)

            def chunk_body(k, _):
                pltpu.async_copy(src_tbl.at[gixs.at[k]], gbuf, gsem).wait()

                def scale_body(g, _):
                    wrow = wvs[k, pl.ds(g * LANES, LANES)]
                    for u in range(LANES):
                        e = g * LANES + u
                        wsplat = jnp.full((LANES,), wrow[u], jnp.float32)
                        for j in range(D // LANES):
                            sl = gbuf[e, pl.ds(j * LANES, LANES)]
                            gbuf[e, pl.ds(j * LANES, LANES)] = sl * wsplat
                    return 0

                lax.fori_loop(0, CHUNK // LANES, scale_body, 0)
                pltpu.sync_copy(gbuf, acc.at[sixs.at[k]], add=True)
                return 0

            lax.fori_loop(0, SUP, chunk_body, 0)
            return 0

        lax.fori_loop(0, N_SUP, sup_body, 0)
        plsc.subcore_barrier()

        for_owned_blocks(
            lambda b: pltpu.sync_copy(acc.at[pl.ds(b, ROW_BLK)],
                                      out_tbl.at[pl.ds(b, ROW_BLK)]))

    @pl.when(cid == 0)
    def _():
        edge_pass(u_in, i_in, ua_out, cols2d, rows2d)

    @pl.when(cid == 1)
    def _():
        edge_pass(i_in, u_in, ia_out, rows2d, cols2d)


LN_TC_BLK = 400


def _ln_tc_body(u_ref, i_ref, w_ref, b_ref, uo_ref, io_ref):
    w = w_ref[...]
    b = b_ref[...]
    for src, dst in ((u_ref, uo_ref), (i_ref, io_ref)):
        x = src[...]
        mu = jnp.mean(x, axis=-1, keepdims=True)
        var = jnp.mean(jnp.square(x - mu), axis=-1, keepdims=True)
        dst[...] = (x - mu) * jax.lax.rsqrt(var + 1e-5) * w + b


def _ln_tc(u_acc, i_acc, lnw, lnb):
    grid = N_USERS // LN_TC_BLK
    blk = pl.BlockSpec((LN_TC_BLK, D), lambda i: (i, 0))
    rep = pl.BlockSpec((1, D), lambda i: (0, 0))
    return pl.pallas_call(
        _ln_tc_body,
        grid=(grid,),
        in_specs=[blk, blk, rep, rep],
        out_specs=[blk, blk],
        out_shape=[
            jax.ShapeDtypeStruct((N_USERS, D), jnp.float32),
            jax.ShapeDtypeStruct((N_ITEMS, D), jnp.float32),
        ],
    )(u_acc, i_acc, lnw.reshape(1, D), lnb.reshape(1, D))


def kernel(user_emb, item_emb, ui_edge_index, ui_edge_weight, ln_weight, ln_bias):
    rows = ui_edge_index[0]
    cols = ui_edge_index[1]
    pad = E_PAD - E
    zpad_i = jnp.zeros((pad,), jnp.int32)
    zpad_f = jnp.zeros((pad,), jnp.float32)
    rows2d = jnp.concatenate([rows, zpad_i]).reshape(E_PAD // CHUNK, CHUNK)
    cols2d = jnp.concatenate([cols, zpad_i]).reshape(E_PAD // CHUNK, CHUNK)
    w2d = jnp.concatenate([ui_edge_weight, zpad_f]).reshape(E_PAD // CHUNK, CHUNK)

    def step(carry, _):
        u, i = carry
        ua, ia = _sc_layer(u, i, rows2d, cols2d, w2d)
        u_n, i_n = _ln_tc(ua, ia, ln_weight, ln_bias)
        return (u_n, i_n), u_n

    (u2, i2), us = lax.scan(step, (user_emb, item_emb), None, length=N_LAYERS)
    return (jnp.concatenate([user_emb[None], us], axis=0), u2, i2)
